# Initial kernel scaffold; baseline (speedup 1.0000x reference)
import jax
import jax.numpy as jnp
from jax.experimental import pallas as pl


def kernel(node_feats, edge_index, edge_attr, coords, node_batch_vec, W1n, b1n, W2n, b2n, W1e, b1e, W2e, b2e, Wc1, bc1, Wc2, Wcm, bcm, Wne, bne, Wee, bee):
    raise NotImplementedError("write your pallas kernel here")


# speedup vs baseline: 4.9330x; 4.9330x over previous
import functools

import jax
import jax.numpy as jnp
from jax import lax
from jax.experimental import pallas as pl
from jax.experimental.pallas import tpu as pltpu
from jax.experimental.pallas import tpu_sc as plsc

N_NODES = 10000
N_EDGES = 320000
D_NODE = 128
D_EDGE = 16
H = 64
OUT = 64
EMB = 32
N_GRAPHS = 64

TW = 128
PW = 128
CH = 128
NC = 2
NS = 16
NW = NC * NS
N_CHUNKS = N_EDGES // CH
CHUNKS_PER_W = (N_CHUNKS + NW - 1) // NW
ROWS_PER_TILE = 632

EBLK = 2000


def _prep_body(nf_ref, coords_ref, w1es_ref, w1ed_ref, b1e_ref,
               tsrc_ref, tdst_ref):
    nf = nf_ref[...]
    coords = coords_ref[...]
    pad = jnp.zeros((N_NODES, TW - H - 3), jnp.float32)
    p = jnp.dot(nf, w1es_ref[...], preferred_element_type=jnp.float32)
    p = p + b1e_ref[...]
    q = jnp.dot(nf, w1ed_ref[...], preferred_element_type=jnp.float32)
    tsrc_ref[...] = jnp.concatenate([p, coords, pad], axis=1)
    tdst_ref[...] = jnp.concatenate([q, coords, pad], axis=1)


def _prep(nf, coords, w1es, w1ed, b1e_row):
    return pl.pallas_call(
        _prep_body,
        out_shape=[
            jax.ShapeDtypeStruct((N_NODES, TW), jnp.float32),
            jax.ShapeDtypeStruct((N_NODES, TW), jnp.float32),
        ],
    )(nf, coords, w1es, w1ed, b1e_row)


def _gather_body(tsrc_hbm, tdst_hbm, src_hbm, dst_hbm,
                 gsrc_hbm, gdst_hbm,
                 idx_s, idx_d, rows_s, rows_d, sem_s, sem_d):
    wid = lax.axis_index("s") * NC + lax.axis_index("c")

    def step(k, _):
        cid = wid + k * NW

        @pl.when(cid < N_CHUNKS)
        def _():
            base = pl.multiple_of(cid * CH, CH)
            pltpu.sync_copy(src_hbm.at[pl.ds(base, CH)], idx_s)
            pltpu.sync_copy(dst_hbm.at[pl.ds(base, CH)], idx_d)
            ds_ = pltpu.async_copy(tsrc_hbm.at[idx_s], rows_s, sem_s)
            dd_ = pltpu.async_copy(tdst_hbm.at[idx_d], rows_d, sem_d)
            ds_.wait()
            dd_.wait()
            pltpu.sync_copy(rows_s, gsrc_hbm.at[pl.ds(base, CH)])
            pltpu.sync_copy(rows_d, gdst_hbm.at[pl.ds(base, CH)])

        return _

    lax.fori_loop(0, CHUNKS_PER_W, step, None)


def _gather(tsrc, tdst, src, dst):
    mesh = plsc.VectorSubcoreMesh(core_axis_name="c", subcore_axis_name="s")
    k = functools.partial(
        pl.kernel,
        mesh=mesh,
        out_type=[
            jax.ShapeDtypeStruct((N_EDGES, TW), jnp.float32),
            jax.ShapeDtypeStruct((N_EDGES, TW), jnp.float32),
        ],
        scratch_types=[
            pltpu.VMEM((CH,), jnp.int32),
            pltpu.VMEM((CH,), jnp.int32),
            pltpu.VMEM((CH, TW), jnp.float32),
            pltpu.VMEM((CH, TW), jnp.float32),
            pltpu.SemaphoreType.DMA,
            pltpu.SemaphoreType.DMA,
        ],
    )(_gather_body)
    return k(tsrc, tdst, src, dst)


def _edge_body(gs_ref, gd_ref, ea_ref, w1ea_ref, w1r_ref, w2e_ref, b2e_ref,
               wc1_ref, bc1_ref, wc2_ref, wee_ref, bee_ref,
               emb_ref, pay_ref):
    gs = gs_ref[...]
    gd = gd_ref[...]
    bond = gs[:, H:H + 3] - gd[:, H:H + 3]
    radial = jnp.sum(bond * bond, axis=1, keepdims=True)
    pre = (gs[:, :H] + gd[:, :H]
           + jnp.dot(ea_ref[...], w1ea_ref[...],
                     preferred_element_type=jnp.float32)
           + radial * w1r_ref[...])
    hidden = jnp.maximum(pre, 0.0)
    edge_out = jnp.dot(hidden, w2e_ref[...],
                       preferred_element_type=jnp.float32) + b2e_ref[...]
    t = jnp.maximum(
        jnp.dot(edge_out, wc1_ref[...], preferred_element_type=jnp.float32)
        + bc1_ref[...], 0.0)
    e_c = t[:, 0:1] * wc2_ref[0, 0] + t[:, 1:2] * wc2_ref[0, 1]
    bond_n = bond / (jnp.sqrt(radial) + 1.0)
    trans = bond_n * e_c
    emb_ref[...] = jnp.dot(edge_out, wee_ref[...],
                           preferred_element_type=jnp.float32) + bee_ref[...]
    pad = jnp.zeros((EBLK, PW - D_EDGE - 3), jnp.float32)
    pay_ref[...] = jnp.concatenate([edge_out, trans, pad], axis=1)


def _edge(gsrc, gdst, edge_attr, w1ea, w1r_row, w2e, b2e_row,
          wc1, bc1_row, wc2_row, wee, bee_row):
    nblk = N_EDGES // EBLK
    row_spec = lambda w: pl.BlockSpec((EBLK, w), lambda i: (i, 0))
    full = lambda s: pl.BlockSpec(s, lambda i: (0, 0))
    return pl.pallas_call(
        _edge_body,
        grid=(nblk,),
        in_specs=[row_spec(TW), row_spec(TW), row_spec(D_EDGE),
                  full((D_EDGE, H)), full((1, H)), full((H, D_EDGE)),
                  full((1, D_EDGE)), full((D_EDGE, 2)), full((1, 2)),
                  full((1, 2)), full((D_EDGE, EMB)), full((1, EMB))],
        out_specs=[row_spec(EMB), row_spec(PW)],
        out_shape=[
            jax.ShapeDtypeStruct((N_EDGES, EMB), jnp.float32),
            jax.ShapeDtypeStruct((N_EDGES, PW), jnp.float32),
        ],
    )(gsrc, gdst, edge_attr, w1ea, w1r_row, w2e, b2e_row,
      wc1, bc1_row, wc2_row, wee, bee_row)


def _scatter_body(pay_hbm, src_hbm, zeros_hbm, agg_hbm,
                  shared, idx, pay, sem):
    cid = lax.axis_index("c")
    sid = lax.axis_index("s")
    wid = sid * NC + cid
    rbase = pl.multiple_of(
        jnp.minimum(sid * ROWS_PER_TILE, N_NODES - ROWS_PER_TILE), 8)

    pltpu.sync_copy(zeros_hbm.at[pl.ds(rbase, ROWS_PER_TILE)],
                    shared.at[pl.ds(rbase, ROWS_PER_TILE)])
    plsc.subcore_barrier()

    def step(k, _):
        chunk = wid + k * NW

        @pl.when(chunk < N_CHUNKS)
        def _():
            base = pl.multiple_of(chunk * CH, CH)
            pltpu.sync_copy(src_hbm.at[pl.ds(base, CH)], idx)
            d = pltpu.async_copy(pay_hbm.at[pl.ds(base, CH)], pay, sem)
            d.wait()
            pltpu.sync_copy(pay, shared.at[idx], add=True)

        return _

    lax.fori_loop(0, CHUNKS_PER_W, step, None)
    plsc.subcore_barrier()
    obase = cid * N_NODES + rbase
    pltpu.sync_copy(shared.at[pl.ds(rbase, ROWS_PER_TILE)],
                    agg_hbm.at[pl.ds(obase, ROWS_PER_TILE)])


def _scatter(payload, src, zeros_tbl):
    mesh = plsc.VectorSubcoreMesh(core_axis_name="c", subcore_axis_name="s")
    k = functools.partial(
        pl.kernel,
        mesh=mesh,
        out_type=jax.ShapeDtypeStruct((NC * N_NODES, PW), jnp.float32),
        scratch_types=[
            pltpu.MemorySpace.VMEM_SHARED((N_NODES, PW), jnp.float32),
            pltpu.VMEM((CH,), jnp.int32),
            pltpu.VMEM((CH, PW), jnp.float32),
            pltpu.SemaphoreType.DMA,
        ],
    )(_scatter_body)
    return k(payload, src, zeros_tbl)


def _node_body(nf_ref, coords_ref, agg_ref, batch_ref,
               w1na_ref, w1nb_ref, w1nc_ref, b1n_ref, w2n_ref, b2n_ref,
               wcm_ref, bcm_ref, wne_ref, bne_ref,
               nemb_ref, gemb_ref, cout_ref):
    agg = agg_ref[0:N_NODES, :] + agg_ref[N_NODES:2 * N_NODES, :]
    agg_e = agg[:, :D_EDGE]
    agg_c = agg[:, D_EDGE:D_EDGE + 3]
    c2 = coords_ref[...] + agg_c
    coord_out = (c2[:, 0:1] * wcm_ref[0:1, :]
                 + c2[:, 1:2] * wcm_ref[1:2, :]
                 + c2[:, 2:3] * wcm_ref[2:3, :]
                 + bcm_ref[...])
    pre = (jnp.dot(nf_ref[...], w1na_ref[...],
                   preferred_element_type=jnp.float32)
           + jnp.dot(agg_e, w1nb_ref[...],
                     preferred_element_type=jnp.float32)
           + coord_out[:, 0:1] * w1nc_ref[0:1, :]
           + coord_out[:, 1:2] * w1nc_ref[1:2, :]
           + coord_out[:, 2:3] * w1nc_ref[2:3, :]
           + b1n_ref[...])
    h = jnp.maximum(pre, 0.0)
    node_out = jnp.dot(h, w2n_ref[...],
                       preferred_element_type=jnp.float32) + b2n_ref[...]
    node_emb = jnp.dot(node_out, wne_ref[...],
                       preferred_element_type=jnp.float32) + bne_ref[...]
    nemb_ref[...] = node_emb
    cout_ref[...] = coord_out
    gids = lax.broadcasted_iota(jnp.int32, (N_GRAPHS, N_NODES), 0)
    onehot_t = (gids == batch_ref[...]).astype(jnp.float32)
    sums = jnp.dot(onehot_t, node_emb, preferred_element_type=jnp.float32)
    counts = jnp.sum(onehot_t, axis=1, keepdims=True)
    gemb_ref[...] = sums / jnp.maximum(counts, 1.0)


def _node(nf, coords, agg, batch_row, w1na, w1nb, w1nc, b1n_row,
          w2n, b2n_row, wcm, bcm_row, wne, bne_row):
    return pl.pallas_call(
        _node_body,
        out_shape=[
            jax.ShapeDtypeStruct((N_NODES, EMB), jnp.float32),
            jax.ShapeDtypeStruct((N_GRAPHS, EMB), jnp.float32),
            jax.ShapeDtypeStruct((N_NODES, 3), jnp.float32),
        ],
    )(nf, coords, agg, batch_row, w1na, w1nb, w1nc, b1n_row,
      w2n, b2n_row, wcm, bcm_row, wne, bne_row)


def kernel(node_feats, edge_index, edge_attr, coords, node_batch_vec,
           W1n, b1n, W2n, b2n, W1e, b1e, W2e, b2e,
           Wc1, bc1, Wc2, Wcm, bcm, Wne, bne, Wee, bee):
    src = edge_index[0]
    dst = edge_index[1]

    w1es = W1e[:D_NODE]
    w1ed = W1e[D_NODE:2 * D_NODE]
    w1ea = W1e[2 * D_NODE:2 * D_NODE + D_EDGE]
    w1r_row = W1e[2 * D_NODE + D_EDGE:]

    tsrc, tdst = _prep(node_feats, coords, w1es, w1ed, b1e.reshape(1, H))
    gsrc, gdst = _gather(tsrc, tdst, src, dst)
    edge_emb, payload = _edge(
        gsrc, gdst, edge_attr, w1ea, w1r_row, W2e, b2e.reshape(1, D_EDGE),
        Wc1, bc1.reshape(1, 2), Wc2.reshape(1, 2), Wee, bee.reshape(1, EMB))
    zeros_tbl = jnp.zeros((N_NODES, PW), jnp.float32)
    agg = _scatter(payload, src, zeros_tbl)
    node_emb, graph_emb, coord_out = _node(
        node_feats, coords, agg, node_batch_vec.reshape(1, N_NODES),
        W1n[:D_NODE], W1n[D_NODE:D_NODE + D_EDGE],
        W1n[D_NODE + D_EDGE:], b1n.reshape(1, H),
        W2n, b2n.reshape(1, OUT), Wcm, bcm.reshape(1, 3),
        Wne, bne.reshape(1, EMB))
    return (node_emb, edge_emb, graph_emb, coord_out)


# speedup vs baseline: 5.2174x; 1.0577x over previous
import functools

import jax
import jax.numpy as jnp
from jax import lax
from jax.experimental import pallas as pl
from jax.experimental.pallas import tpu as pltpu
from jax.experimental.pallas import tpu_sc as plsc

N_NODES = 10000
N_EDGES = 320000
D_NODE = 128
D_EDGE = 16
H = 64
OUT = 64
EMB = 32
N_GRAPHS = 64

TW = 128
PW = 128
CH = 128
NC = 2
NS = 16
NW = NC * NS
N_CHUNKS = N_EDGES // CH
CHUNKS_PER_W = (N_CHUNKS + NW - 1) // NW
ROWS_PER_TILE = 632

EBLK = 2000


def _prep_body(nf_ref, coords_ref, w1es_ref, w1ed_ref, b1e_ref,
               tsrc_ref, tdst_ref):
    nf = nf_ref[...]
    coords = coords_ref[...]
    pad = jnp.zeros((N_NODES, TW - H - 3), jnp.float32)
    p = jnp.dot(nf, w1es_ref[...], preferred_element_type=jnp.float32)
    p = p + b1e_ref[...]
    q = jnp.dot(nf, w1ed_ref[...], preferred_element_type=jnp.float32)
    tsrc_ref[...] = jnp.concatenate([p, coords, pad], axis=1)
    tdst_ref[...] = jnp.concatenate([q, coords, pad], axis=1)


def _prep(nf, coords, w1es, w1ed, b1e_row):
    return pl.pallas_call(
        _prep_body,
        out_shape=[
            jax.ShapeDtypeStruct((N_NODES, TW), jnp.float32),
            jax.ShapeDtypeStruct((N_NODES, TW), jnp.float32),
        ],
    )(nf, coords, w1es, w1ed, b1e_row)


def _gather_body(tsrc_hbm, tdst_hbm, src_hbm, dst_hbm,
                 gsrc_hbm, gdst_hbm,
                 shared_t, idx_v, rows_v, sem):
    cid = lax.axis_index("c")
    sid = lax.axis_index("s")
    rbase = pl.multiple_of(
        jnp.minimum(sid * ROWS_PER_TILE, N_NODES - ROWS_PER_TILE), 8)

    @pl.when(cid == 0)
    def _():
        pltpu.sync_copy(tsrc_hbm.at[pl.ds(rbase, ROWS_PER_TILE)],
                        shared_t.at[pl.ds(rbase, ROWS_PER_TILE)])

    @pl.when(cid == 1)
    def _():
        pltpu.sync_copy(tdst_hbm.at[pl.ds(rbase, ROWS_PER_TILE)],
                        shared_t.at[pl.ds(rbase, ROWS_PER_TILE)])

    plsc.subcore_barrier()

    def step(k, _):
        chunk = sid + k * NS

        @pl.when(chunk < N_CHUNKS)
        def _():
            base = pl.multiple_of(chunk * CH, CH)

            @pl.when(cid == 0)
            def _():
                pltpu.sync_copy(src_hbm.at[pl.ds(base, CH)], idx_v)
                pltpu.async_copy(shared_t.at[idx_v], rows_v, sem).wait()
                pltpu.sync_copy(rows_v, gsrc_hbm.at[pl.ds(base, CH)])

            @pl.when(cid == 1)
            def _():
                pltpu.sync_copy(dst_hbm.at[pl.ds(base, CH)], idx_v)
                pltpu.async_copy(shared_t.at[idx_v], rows_v, sem).wait()
                pltpu.sync_copy(rows_v, gdst_hbm.at[pl.ds(base, CH)])

        return _

    lax.fori_loop(0, (N_CHUNKS + NS - 1) // NS, step, None)


def _gather(tsrc, tdst, src, dst):
    mesh = plsc.VectorSubcoreMesh(core_axis_name="c", subcore_axis_name="s")
    k = functools.partial(
        pl.kernel,
        mesh=mesh,
        out_type=[
            jax.ShapeDtypeStruct((N_EDGES, TW), jnp.float32),
            jax.ShapeDtypeStruct((N_EDGES, TW), jnp.float32),
        ],
        scratch_types=[
            pltpu.MemorySpace.VMEM_SHARED((N_NODES, TW), jnp.float32),
            pltpu.VMEM((CH,), jnp.int32),
            pltpu.VMEM((CH, TW), jnp.float32),
            pltpu.SemaphoreType.DMA,
        ],
    )(_gather_body)
    return k(tsrc, tdst, src, dst)


def _edge_body(gs_ref, gd_ref, ea_ref, w1ea_ref, w1r_ref, w2e_ref, b2e_ref,
               wc1_ref, bc1_ref, wc2_ref, wee_ref, bee_ref,
               emb_ref, pay_ref):
    gs = gs_ref[...].astype(jnp.float32)
    gd = gd_ref[...].astype(jnp.float32)
    bond = gs[:, H:H + 3] - gd[:, H:H + 3]
    radial = jnp.sum(bond * bond, axis=1, keepdims=True)
    pre = (gs[:, :H] + gd[:, :H]
           + jnp.dot(ea_ref[...], w1ea_ref[...],
                     preferred_element_type=jnp.float32)
           + radial * w1r_ref[...])
    hidden = jnp.maximum(pre, 0.0)
    edge_out = jnp.dot(hidden, w2e_ref[...],
                       preferred_element_type=jnp.float32) + b2e_ref[...]
    t = jnp.maximum(
        jnp.dot(edge_out, wc1_ref[...], preferred_element_type=jnp.float32)
        + bc1_ref[...], 0.0)
    e_c = t[:, 0:1] * wc2_ref[0, 0] + t[:, 1:2] * wc2_ref[0, 1]
    bond_n = bond / (jnp.sqrt(radial) + 1.0)
    trans = bond_n * e_c
    emb_ref[...] = jnp.dot(edge_out, wee_ref[...],
                           preferred_element_type=jnp.float32) + bee_ref[...]
    pad = jnp.zeros((EBLK, PW - D_EDGE - 3), jnp.float32)
    pay_ref[...] = jnp.concatenate([edge_out, trans, pad], axis=1)


def _edge(gsrc, gdst, edge_attr, w1ea, w1r_row, w2e, b2e_row,
          wc1, bc1_row, wc2_row, wee, bee_row):
    nblk = N_EDGES // EBLK
    row_spec = lambda w: pl.BlockSpec((EBLK, w), lambda i: (i, 0))
    full = lambda s: pl.BlockSpec(s, lambda i: (0, 0))
    return pl.pallas_call(
        _edge_body,
        grid=(nblk,),
        in_specs=[row_spec(TW), row_spec(TW), row_spec(D_EDGE),
                  full((D_EDGE, H)), full((1, H)), full((H, D_EDGE)),
                  full((1, D_EDGE)), full((D_EDGE, 2)), full((1, 2)),
                  full((1, 2)), full((D_EDGE, EMB)), full((1, EMB))],
        out_specs=[row_spec(EMB), row_spec(PW)],
        out_shape=[
            jax.ShapeDtypeStruct((N_EDGES, EMB), jnp.float32),
            jax.ShapeDtypeStruct((N_EDGES, PW), jnp.float32),
        ],
    )(gsrc, gdst, edge_attr, w1ea, w1r_row, w2e, b2e_row,
      wc1, bc1_row, wc2_row, wee, bee_row)


def _scatter_body(pay_hbm, src_hbm, zeros_hbm, agg_hbm,
                  shared, idx, pay, sem):
    cid = lax.axis_index("c")
    sid = lax.axis_index("s")
    wid = sid * NC + cid
    rbase = pl.multiple_of(
        jnp.minimum(sid * ROWS_PER_TILE, N_NODES - ROWS_PER_TILE), 8)

    pltpu.sync_copy(zeros_hbm.at[pl.ds(rbase, ROWS_PER_TILE)],
                    shared.at[pl.ds(rbase, ROWS_PER_TILE)])
    plsc.subcore_barrier()

    def step(k, _):
        chunk = wid + k * NW

        @pl.when(chunk < N_CHUNKS)
        def _():
            base = pl.multiple_of(chunk * CH, CH)
            pltpu.sync_copy(src_hbm.at[pl.ds(base, CH)], idx)
            d = pltpu.async_copy(pay_hbm.at[pl.ds(base, CH)], pay, sem)
            d.wait()
            pltpu.sync_copy(pay, shared.at[idx], add=True)

        return _

    lax.fori_loop(0, CHUNKS_PER_W, step, None)
    plsc.subcore_barrier()
    obase = cid * N_NODES + rbase
    pltpu.sync_copy(shared.at[pl.ds(rbase, ROWS_PER_TILE)],
                    agg_hbm.at[pl.ds(obase, ROWS_PER_TILE)])


def _scatter(payload, src, zeros_tbl):
    mesh = plsc.VectorSubcoreMesh(core_axis_name="c", subcore_axis_name="s")
    k = functools.partial(
        pl.kernel,
        mesh=mesh,
        out_type=jax.ShapeDtypeStruct((NC * N_NODES, PW), jnp.float32),
        scratch_types=[
            pltpu.MemorySpace.VMEM_SHARED((N_NODES, PW), jnp.float32),
            pltpu.VMEM((CH,), jnp.int32),
            pltpu.VMEM((CH, PW), jnp.float32),
            pltpu.SemaphoreType.DMA,
        ],
    )(_scatter_body)
    return k(payload, src, zeros_tbl)


def _node_body(nf_ref, coords_ref, agg_ref, batch_ref,
               w1na_ref, w1nb_ref, w1nc_ref, b1n_ref, w2n_ref, b2n_ref,
               wcm_ref, bcm_ref, wne_ref, bne_ref,
               nemb_ref, gemb_ref, cout_ref):
    agg = agg_ref[0:N_NODES, :] + agg_ref[N_NODES:2 * N_NODES, :]
    agg_e = agg[:, :D_EDGE]
    agg_c = agg[:, D_EDGE:D_EDGE + 3]
    c2 = coords_ref[...] + agg_c
    coord_out = (c2[:, 0:1] * wcm_ref[0:1, :]
                 + c2[:, 1:2] * wcm_ref[1:2, :]
                 + c2[:, 2:3] * wcm_ref[2:3, :]
                 + bcm_ref[...])
    pre = (jnp.dot(nf_ref[...], w1na_ref[...],
                   preferred_element_type=jnp.float32)
           + jnp.dot(agg_e, w1nb_ref[...],
                     preferred_element_type=jnp.float32)
           + coord_out[:, 0:1] * w1nc_ref[0:1, :]
           + coord_out[:, 1:2] * w1nc_ref[1:2, :]
           + coord_out[:, 2:3] * w1nc_ref[2:3, :]
           + b1n_ref[...])
    h = jnp.maximum(pre, 0.0)
    node_out = jnp.dot(h, w2n_ref[...],
                       preferred_element_type=jnp.float32) + b2n_ref[...]
    node_emb = jnp.dot(node_out, wne_ref[...],
                       preferred_element_type=jnp.float32) + bne_ref[...]
    nemb_ref[...] = node_emb
    cout_ref[...] = coord_out
    gids = lax.broadcasted_iota(jnp.int32, (N_GRAPHS, N_NODES), 0)
    onehot_t = (gids == batch_ref[...]).astype(jnp.float32)
    sums = jnp.dot(onehot_t, node_emb, preferred_element_type=jnp.float32)
    counts = jnp.sum(onehot_t, axis=1, keepdims=True)
    gemb_ref[...] = sums / jnp.maximum(counts, 1.0)


def _node(nf, coords, agg, batch_row, w1na, w1nb, w1nc, b1n_row,
          w2n, b2n_row, wcm, bcm_row, wne, bne_row):
    return pl.pallas_call(
        _node_body,
        out_shape=[
            jax.ShapeDtypeStruct((N_NODES, EMB), jnp.float32),
            jax.ShapeDtypeStruct((N_GRAPHS, EMB), jnp.float32),
            jax.ShapeDtypeStruct((N_NODES, 3), jnp.float32),
        ],
    )(nf, coords, agg, batch_row, w1na, w1nb, w1nc, b1n_row,
      w2n, b2n_row, wcm, bcm_row, wne, bne_row)


def kernel(node_feats, edge_index, edge_attr, coords, node_batch_vec,
           W1n, b1n, W2n, b2n, W1e, b1e, W2e, b2e,
           Wc1, bc1, Wc2, Wcm, bcm, Wne, bne, Wee, bee):
    src = edge_index[0]
    dst = edge_index[1]

    w1es = W1e[:D_NODE]
    w1ed = W1e[D_NODE:2 * D_NODE]
    w1ea = W1e[2 * D_NODE:2 * D_NODE + D_EDGE]
    w1r_row = W1e[2 * D_NODE + D_EDGE:]

    tsrc, tdst = _prep(node_feats, coords, w1es, w1ed, b1e.reshape(1, H))
    gsrc, gdst = _gather(tsrc, tdst, src, dst)
    edge_emb, payload = _edge(
        gsrc, gdst, edge_attr, w1ea, w1r_row, W2e, b2e.reshape(1, D_EDGE),
        Wc1, bc1.reshape(1, 2), Wc2.reshape(1, 2), Wee, bee.reshape(1, EMB))
    zeros_tbl = jnp.zeros((N_NODES, PW), jnp.float32)
    agg = _scatter(payload, src, zeros_tbl)
    node_emb, graph_emb, coord_out = _node(
        node_feats, coords, agg, node_batch_vec.reshape(1, N_NODES),
        W1n[:D_NODE], W1n[D_NODE:D_NODE + D_EDGE],
        W1n[D_NODE + D_EDGE:], b1n.reshape(1, H),
        W2n, b2n.reshape(1, OUT), Wcm, bcm.reshape(1, 3),
        Wne, bne.reshape(1, EMB))
    return (node_emb, edge_emb, graph_emb, coord_out)


# speedup vs baseline: 6.3888x; 1.2245x over previous
import functools

import jax
import jax.numpy as jnp
from jax import lax
from jax.experimental import pallas as pl
from jax.experimental.pallas import tpu as pltpu
from jax.experimental.pallas import tpu_sc as plsc

N_NODES = 10000
N_EDGES = 320000
D_NODE = 128
D_EDGE = 16
H = 64
OUT = 64
EMB = 32
N_GRAPHS = 64

TW = 128
PW = 128
CH = 128
NC = 2
NS = 16
NW = NC * NS
N_CHUNKS = N_EDGES // CH
CPT_G = 160
CPT_S = 80
IDXROWS = 80
NBUF = 2
ROWS_PER_TILE = 632

EBLK = 2000
NBLK = N_EDGES // EBLK


def _prep_body(nf_ref, coords_ref, w1es_ref, w1ed_ref, b1e_ref, t2_ref):
    nf = nf_ref[...]
    coords = coords_ref[...]
    pad = jnp.zeros((N_NODES, TW - H - 3), jnp.float32)
    p = jnp.dot(nf, w1es_ref[...], preferred_element_type=jnp.float32)
    p = p + b1e_ref[...]
    q = jnp.dot(nf, w1ed_ref[...], preferred_element_type=jnp.float32)
    t2_ref[0:N_NODES, :] = jnp.concatenate([p, coords, pad], axis=1)
    t2_ref[N_NODES:2 * N_NODES, :] = jnp.concatenate([q, coords, pad], axis=1)


def _prep(nf, coords, w1es, w1ed, b1e_row):
    return pl.pallas_call(
        _prep_body,
        out_shape=jax.ShapeDtypeStruct((2 * N_NODES, TW), jnp.float32),
    )(nf, coords, w1es, w1ed, b1e_row)


def _gather_body(t2_hbm, sd_hbm, g2_hbm,
                 shared_t, idx_all, b0, b1, gs0, gs1, ws0, ws1):
    bufs = (b0, b1)
    gsem = (gs0, gs1)
    wsem = (ws0, ws1)
    cid = lax.axis_index("c")
    sid = lax.axis_index("s")

    rbase = pl.multiple_of(
        jnp.minimum(sid * ROWS_PER_TILE, N_NODES - ROWS_PER_TILE), 8)
    pltpu.sync_copy(t2_hbm.at[pl.ds(cid * N_NODES + rbase, ROWS_PER_TILE)],
                    shared_t.at[pl.ds(rbase, ROWS_PER_TILE)])

    cbase = sid * CPT_G
    valid = jnp.minimum(CPT_G, N_CHUNKS - cbase)
    irow0 = cid * (NS * CPT_G) + cbase
    pltpu.sync_copy(sd_hbm.at[pl.ds(irow0, IDXROWS)], idx_all)
    plsc.subcore_barrier()

    ebase = cid * N_EDGES
    half_g = IDXROWS // NBUF

    def group(g, _):
        @pl.when(g == half_g)
        def _():
            pltpu.sync_copy(sd_hbm.at[pl.ds(irow0 + IDXROWS, IDXROWS)],
                            idx_all)

        for i in range(NBUF):
            j = g * NBUF + i

            @pl.when(j < valid)
            def _():
                @pl.when(g > 0)
                def _():
                    prev = ebase + (cbase + j - NBUF) * CH
                    pltpu.make_async_copy(
                        bufs[i], g2_hbm.at[pl.ds(prev, CH)], wsem[i]).wait()

                pltpu.async_copy(shared_t.at[idx_all.at[lax.rem(j, IDXROWS)]],
                                 bufs[i], gsem[i])

        for i in range(NBUF):
            j = g * NBUF + i

            @pl.when(j < valid)
            def _():
                pltpu.make_async_copy(
                    shared_t.at[idx_all.at[lax.rem(j, IDXROWS)]],
                    bufs[i], gsem[i]).wait()
                off = ebase + (cbase + j) * CH
                pltpu.async_copy(bufs[i], g2_hbm.at[pl.ds(off, CH)], wsem[i])

        return _

    lax.fori_loop(0, CPT_G // NBUF, group, None)

    for i in range(NBUF):
        n_i = (valid + (NBUF - 1) - i) // NBUF

        @pl.when(valid > i)
        def _():
            last = ebase + (cbase + (n_i - 1) * NBUF + i) * CH
            pltpu.make_async_copy(
                bufs[i], g2_hbm.at[pl.ds(last, CH)], wsem[i]).wait()


def _gather(t2, sd):
    mesh = plsc.VectorSubcoreMesh(core_axis_name="c", subcore_axis_name="s")
    k = functools.partial(
        pl.kernel,
        mesh=mesh,
        out_type=jax.ShapeDtypeStruct((2 * N_EDGES, TW), jnp.float32),
        scratch_types=[
            pltpu.MemorySpace.VMEM_SHARED((N_NODES, TW), jnp.float32),
            pltpu.VMEM((IDXROWS, CH), jnp.int32),
        ] + [pltpu.VMEM((CH, TW), jnp.float32)] * NBUF
          + [pltpu.SemaphoreType.DMA] * (2 * NBUF),
    )(_gather_body)
    return k(t2, sd)


def _edge_body(gs_ref, gd_ref, ea_ref, w1ea_ref, w1r_ref, w2e_ref, b2e_ref,
               wc1_ref, bc1_ref, wc2_ref, wee_ref, bee_ref,
               emb_ref, pay_ref):
    gs = gs_ref[...]
    gd = gd_ref[...]
    bond = gs[:, H:H + 3] - gd[:, H:H + 3]
    radial = jnp.sum(bond * bond, axis=1, keepdims=True)
    pre = (gs[:, :H] + gd[:, :H]
           + jnp.dot(ea_ref[...], w1ea_ref[...],
                     preferred_element_type=jnp.float32)
           + radial * w1r_ref[...])
    hidden = jnp.maximum(pre, 0.0)
    edge_out = jnp.dot(hidden, w2e_ref[...],
                       preferred_element_type=jnp.float32) + b2e_ref[...]
    t = jnp.maximum(
        jnp.dot(edge_out, wc1_ref[...], preferred_element_type=jnp.float32)
        + bc1_ref[...], 0.0)
    e_c = t[:, 0:1] * wc2_ref[0, 0] + t[:, 1:2] * wc2_ref[0, 1]
    bond_n = bond / (jnp.sqrt(radial) + 1.0)
    trans = bond_n * e_c
    emb_ref[...] = jnp.dot(edge_out, wee_ref[...],
                           preferred_element_type=jnp.float32) + bee_ref[...]
    pad = jnp.zeros((EBLK, PW - D_EDGE - 3), jnp.float32)
    pay_ref[...] = jnp.concatenate([edge_out, trans, pad], axis=1)


def _edge(g2, edge_attr, w1ea, w1r_row, w2e, b2e_row,
          wc1, bc1_row, wc2_row, wee, bee_row):
    row_spec = lambda w: pl.BlockSpec((EBLK, w), lambda i: (i, 0))
    full = lambda s: pl.BlockSpec(s, lambda i: (0, 0))
    return pl.pallas_call(
        _edge_body,
        grid=(NBLK,),
        in_specs=[pl.BlockSpec((EBLK, TW), lambda i: (i, 0)),
                  pl.BlockSpec((EBLK, TW), lambda i: (NBLK + i, 0)),
                  row_spec(D_EDGE),
                  full((D_EDGE, H)), full((1, H)), full((H, D_EDGE)),
                  full((1, D_EDGE)), full((D_EDGE, 2)), full((1, 2)),
                  full((1, 2)), full((D_EDGE, EMB)), full((1, EMB))],
        out_specs=[row_spec(EMB), row_spec(PW)],
        out_shape=[
            jax.ShapeDtypeStruct((N_EDGES, EMB), jnp.float32),
            jax.ShapeDtypeStruct((N_EDGES, PW), jnp.float32),
        ],
    )(g2, g2, edge_attr, w1ea, w1r_row, w2e, b2e_row,
      wc1, bc1_row, wc2_row, wee, bee_row)


def _scatter_body(pay_hbm, sd_hbm, zeros_hbm, agg_hbm,
                  shared, idx_all, b0, b1, ps0, ps1, ss0, ss1):
    bufs = (b0, b1)
    psem = (ps0, ps1)
    ssem = (ss0, ss1)
    cid = lax.axis_index("c")
    sid = lax.axis_index("s")
    wid = sid * NC + cid

    rbase = pl.multiple_of(
        jnp.minimum(sid * ROWS_PER_TILE, N_NODES - ROWS_PER_TILE), 8)
    pltpu.sync_copy(zeros_hbm.at[pl.ds(rbase, ROWS_PER_TILE)],
                    shared.at[pl.ds(rbase, ROWS_PER_TILE)])

    cbase = wid * CPT_S
    valid = jnp.clip(N_CHUNKS - cbase, 0, CPT_S)
    pltpu.sync_copy(sd_hbm.at[pl.ds(cbase, CPT_S)], idx_all)
    plsc.subcore_barrier()

    def group(g, _):
        for i in range(NBUF):
            j = g * NBUF + i

            @pl.when(j < valid)
            def _():
                @pl.when(g > 0)
                def _():
                    pltpu.make_async_copy(
                        bufs[i], shared.at[idx_all.at[j - NBUF]],
                        ssem[i]).wait()

                off = (cbase + j) * CH
                pltpu.async_copy(pay_hbm.at[pl.ds(off, CH)], bufs[i], psem[i])

        for i in range(NBUF):
            j = g * NBUF + i

            @pl.when(j < valid)
            def _():
                off = (cbase + j) * CH
                pltpu.make_async_copy(
                    pay_hbm.at[pl.ds(off, CH)], bufs[i], psem[i]).wait()
                pltpu.async_copy(bufs[i], shared.at[idx_all.at[j]],
                                 ssem[i], add=True)

        return _

    lax.fori_loop(0, CPT_S // NBUF, group, None)

    for i in range(NBUF):
        n_i = (valid + (NBUF - 1) - i) // NBUF

        @pl.when(valid > i)
        def _():
            last_j = (n_i - 1) * NBUF + i
            pltpu.make_async_copy(
                bufs[i], shared.at[idx_all.at[last_j]], ssem[i]).wait()

    plsc.subcore_barrier()
    pltpu.sync_copy(shared.at[pl.ds(rbase, ROWS_PER_TILE)],
                    agg_hbm.at[pl.ds(cid * N_NODES + rbase, ROWS_PER_TILE)])


def _scatter(payload, sd, zeros_tbl):
    mesh = plsc.VectorSubcoreMesh(core_axis_name="c", subcore_axis_name="s")
    k = functools.partial(
        pl.kernel,
        mesh=mesh,
        out_type=jax.ShapeDtypeStruct((NC * N_NODES, PW), jnp.float32),
        scratch_types=[
            pltpu.MemorySpace.VMEM_SHARED((N_NODES, PW), jnp.float32),
            pltpu.VMEM((CPT_S, CH), jnp.int32),
        ] + [pltpu.VMEM((CH, PW), jnp.float32)] * NBUF
          + [pltpu.SemaphoreType.DMA] * (2 * NBUF),
    )(_scatter_body)
    return k(payload, sd, zeros_tbl)


def _node_body(nf_ref, coords_ref, agg_ref, batch_ref,
               w1na_ref, w1nb_ref, w1nc_ref, b1n_ref, w2n_ref, b2n_ref,
               wcm_ref, bcm_ref, wne_ref, bne_ref,
               nemb_ref, gemb_ref, cout_ref):
    agg = agg_ref[0:N_NODES, :] + agg_ref[N_NODES:2 * N_NODES, :]
    agg_e = agg[:, :D_EDGE]
    agg_c = agg[:, D_EDGE:D_EDGE + 3]
    c2 = coords_ref[...] + agg_c
    coord_out = (c2[:, 0:1] * wcm_ref[0:1, :]
                 + c2[:, 1:2] * wcm_ref[1:2, :]
                 + c2[:, 2:3] * wcm_ref[2:3, :]
                 + bcm_ref[...])
    pre = (jnp.dot(nf_ref[...], w1na_ref[...],
                   preferred_element_type=jnp.float32)
           + jnp.dot(agg_e, w1nb_ref[...],
                     preferred_element_type=jnp.float32)
           + coord_out[:, 0:1] * w1nc_ref[0:1, :]
           + coord_out[:, 1:2] * w1nc_ref[1:2, :]
           + coord_out[:, 2:3] * w1nc_ref[2:3, :]
           + b1n_ref[...])
    h = jnp.maximum(pre, 0.0)
    node_out = jnp.dot(h, w2n_ref[...],
                       preferred_element_type=jnp.float32) + b2n_ref[...]
    node_emb = jnp.dot(node_out, wne_ref[...],
                       preferred_element_type=jnp.float32) + bne_ref[...]
    nemb_ref[...] = node_emb
    cout_ref[...] = coord_out
    gids = lax.broadcasted_iota(jnp.int32, (N_GRAPHS, N_NODES), 0)
    onehot_t = (gids == batch_ref[...]).astype(jnp.float32)
    sums = jnp.dot(onehot_t, node_emb, preferred_element_type=jnp.float32)
    counts = jnp.sum(onehot_t, axis=1, keepdims=True)
    gemb_ref[...] = sums / jnp.maximum(counts, 1.0)


def _node(nf, coords, agg, batch_row, w1na, w1nb, w1nc, b1n_row,
          w2n, b2n_row, wcm, bcm_row, wne, bne_row):
    return pl.pallas_call(
        _node_body,
        out_shape=[
            jax.ShapeDtypeStruct((N_NODES, EMB), jnp.float32),
            jax.ShapeDtypeStruct((N_GRAPHS, EMB), jnp.float32),
            jax.ShapeDtypeStruct((N_NODES, 3), jnp.float32),
        ],
    )(nf, coords, agg, batch_row, w1na, w1nb, w1nc, b1n_row,
      w2n, b2n_row, wcm, bcm_row, wne, bne_row)


def kernel(node_feats, edge_index, edge_attr, coords, node_batch_vec,
           W1n, b1n, W2n, b2n, W1e, b1e, W2e, b2e,
           Wc1, bc1, Wc2, Wcm, bcm, Wne, bne, Wee, bee):
    w1es = W1e[:D_NODE]
    w1ed = W1e[D_NODE:2 * D_NODE]
    w1ea = W1e[2 * D_NODE:2 * D_NODE + D_EDGE]
    w1r_row = W1e[2 * D_NODE + D_EDGE:]

    sd = jnp.pad(edge_index.reshape(2, N_CHUNKS, CH),
                 ((0, 0), (0, NS * CPT_G - N_CHUNKS), (0, 0))
                 ).reshape(2 * NS * CPT_G, CH)

    t2 = _prep(node_feats, coords, w1es, w1ed, b1e.reshape(1, H))
    g2 = _gather(t2, sd)
    edge_emb, payload = _edge(
        g2, edge_attr, w1ea, w1r_row, W2e, b2e.reshape(1, D_EDGE),
        Wc1, bc1.reshape(1, 2), Wc2.reshape(1, 2), Wee, bee.reshape(1, EMB))
    zeros_tbl = jnp.zeros((N_NODES, PW), jnp.float32)
    agg = _scatter(payload, sd, zeros_tbl)
    node_emb, graph_emb, coord_out = _node(
        node_feats, coords, agg, node_batch_vec.reshape(1, N_NODES),
        W1n[:D_NODE], W1n[D_NODE:D_NODE + D_EDGE],
        W1n[D_NODE + D_EDGE:], b1n.reshape(1, H),
        W2n, b2n.reshape(1, OUT), Wcm, bcm.reshape(1, 3),
        Wne, bne.reshape(1, EMB))
    return (node_emb, edge_emb, graph_emb, coord_out)


# speedup vs baseline: 6.7065x; 1.0497x over previous
import functools

import jax
import jax.numpy as jnp
from jax import lax
from jax.experimental import pallas as pl
from jax.experimental.pallas import tpu as pltpu
from jax.experimental.pallas import tpu_sc as plsc

N_NODES = 10000
N_EDGES = 320000
D_NODE = 128
D_EDGE = 16
H = 64
OUT = 64
EMB = 32
N_GRAPHS = 64

TW = 128
PW = 128
CH = 128
NC = 2
NS = 16
NW = NC * NS
N_CHUNKS = N_EDGES // CH
SD_SIDE = 2560
C0_A, NCAP_A = 0, 1280
C0_B, NCAP_B = 1280, 1220
CPT_G = 80
CPT_S = 40
NBUF = 2
ROWS_PER_TILE = 632

EBLK = 2560


def _prep_body(nf_ref, coords_ref, w1es_ref, w1ed_ref, b1e_ref, t2_ref):
    nf = nf_ref[...]
    coords = coords_ref[...]
    pad = jnp.zeros((N_NODES, TW - H - 3), jnp.float32)
    p = jnp.dot(nf, w1es_ref[...], preferred_element_type=jnp.float32)
    p = p + b1e_ref[...]
    q = jnp.dot(nf, w1ed_ref[...], preferred_element_type=jnp.float32)
    t2_ref[0:N_NODES, :] = jnp.concatenate([p, coords, pad], axis=1)
    t2_ref[N_NODES:2 * N_NODES, :] = jnp.concatenate([q, coords, pad], axis=1)


def _prep(nf, coords, w1es, w1ed, b1e_row):
    return pl.pallas_call(
        _prep_body,
        out_shape=jax.ShapeDtypeStruct((2 * N_NODES, TW), jnp.float32),
    )(nf, coords, w1es, w1ed, b1e_row)


def _gather_body(c0, ncap, t2_hbm, sd_hbm, g2_hbm,
                 shared_t, idx_all, b0, b1, gs0, gs1, ws0, ws1):
    bufs = (b0, b1)
    gsem = (gs0, gs1)
    wsem = (ws0, ws1)
    cid = lax.axis_index("c")
    sid = lax.axis_index("s")

    rbase = pl.multiple_of(
        jnp.minimum(sid * ROWS_PER_TILE, N_NODES - ROWS_PER_TILE), 8)
    pltpu.sync_copy(t2_hbm.at[pl.ds(cid * N_NODES + rbase, ROWS_PER_TILE)],
                    shared_t.at[pl.ds(rbase, ROWS_PER_TILE)])

    cbase = sid * CPT_G
    valid = jnp.clip(ncap - cbase, 0, CPT_G)
    irow0 = cid * SD_SIDE + c0 + cbase
    pltpu.sync_copy(sd_hbm.at[pl.ds(irow0, CPT_G)], idx_all)
    plsc.subcore_barrier()

    ebase = cid * (ncap * CH)

    def group(g, _):
        for i in range(NBUF):
            j = g * NBUF + i

            @pl.when(j < valid)
            def _():
                @pl.when(g > 0)
                def _():
                    prev = ebase + (cbase + j - NBUF) * CH
                    pltpu.make_async_copy(
                        bufs[i], g2_hbm.at[pl.ds(prev, CH)], wsem[i]).wait()

                pltpu.async_copy(shared_t.at[idx_all.at[j]], bufs[i], gsem[i])

        for i in range(NBUF):
            j = g * NBUF + i

            @pl.when(j < valid)
            def _():
                pltpu.make_async_copy(
                    shared_t.at[idx_all.at[j]], bufs[i], gsem[i]).wait()
                off = ebase + (cbase + j) * CH
                pltpu.async_copy(bufs[i], g2_hbm.at[pl.ds(off, CH)], wsem[i])

        return _

    lax.fori_loop(0, CPT_G // NBUF, group, None)

    for i in range(NBUF):
        n_i = (valid + (NBUF - 1) - i) // NBUF

        @pl.when(valid > i)
        def _():
            last = ebase + (cbase + (n_i - 1) * NBUF + i) * CH
            pltpu.make_async_copy(
                bufs[i], g2_hbm.at[pl.ds(last, CH)], wsem[i]).wait()


def _gather(t2, sd, c0, ncap):
    mesh = plsc.VectorSubcoreMesh(core_axis_name="c", subcore_axis_name="s")
    k = functools.partial(
        pl.kernel,
        mesh=mesh,
        out_type=jax.ShapeDtypeStruct((2 * ncap * CH, TW), jnp.float32),
        scratch_types=[
            pltpu.MemorySpace.VMEM_SHARED((N_NODES, TW), jnp.float32),
            pltpu.VMEM((CPT_G, CH), jnp.int32),
        ] + [pltpu.VMEM((CH, TW), jnp.float32)] * NBUF
          + [pltpu.SemaphoreType.DMA] * (2 * NBUF),
    )(functools.partial(_gather_body, c0, ncap))
    return k(t2, sd)


def _edge_body(gs_ref, gd_ref, ea_ref, w1ea_ref, w1r_ref, w2e_ref, b2e_ref,
               wc1_ref, bc1_ref, wc2_ref, wee_ref, bee_ref, emb_in_ref,
               emb_ref, pay_ref):
    del emb_in_ref
    gs = gs_ref[...]
    gd = gd_ref[...]
    bond = gs[:, H:H + 3] - gd[:, H:H + 3]
    radial = jnp.sum(bond * bond, axis=1, keepdims=True)
    pre = (gs[:, :H] + gd[:, :H]
           + jnp.dot(ea_ref[...], w1ea_ref[...],
                     preferred_element_type=jnp.float32)
           + radial * w1r_ref[...])
    hidden = jnp.maximum(pre, 0.0)
    edge_out = jnp.dot(hidden, w2e_ref[...],
                       preferred_element_type=jnp.float32) + b2e_ref[...]
    t = jnp.maximum(
        jnp.dot(edge_out, wc1_ref[...], preferred_element_type=jnp.float32)
        + bc1_ref[...], 0.0)
    e_c = t[:, 0:1] * wc2_ref[0, 0] + t[:, 1:2] * wc2_ref[0, 1]
    bond_n = bond / (jnp.sqrt(radial) + 1.0)
    trans = bond_n * e_c
    emb_ref[...] = jnp.dot(edge_out, wee_ref[...],
                           preferred_element_type=jnp.float32) + bee_ref[...]
    pad = jnp.zeros((EBLK, PW - D_EDGE - 3), jnp.float32)
    pay_ref[...] = jnp.concatenate([edge_out, trans, pad], axis=1)


def _edge(g2, edge_attr, w1ea, w1r_row, w2e, b2e_row,
          wc1, bc1_row, wc2_row, wee, bee_row, emb_in, nblk, blk0):
    full = lambda s: pl.BlockSpec(s, lambda i: (0, 0))
    return pl.pallas_call(
        _edge_body,
        grid=(nblk,),
        in_specs=[pl.BlockSpec((EBLK, TW), lambda i: (i, 0)),
                  pl.BlockSpec((EBLK, TW), lambda i, n=nblk: (n + i, 0)),
                  pl.BlockSpec((EBLK, D_EDGE), lambda i, b=blk0: (b + i, 0)),
                  full((D_EDGE, H)), full((1, H)), full((H, D_EDGE)),
                  full((1, D_EDGE)), full((D_EDGE, 2)), full((1, 2)),
                  full((1, 2)), full((D_EDGE, EMB)), full((1, EMB)),
                  pl.BlockSpec((8, EMB), lambda i: (0, 0))],
        out_specs=[pl.BlockSpec((EBLK, EMB), lambda i, b=blk0: (b + i, 0)),
                   pl.BlockSpec((EBLK, PW), lambda i: (i, 0))],
        out_shape=[
            jax.ShapeDtypeStruct((N_EDGES, EMB), jnp.float32),
            jax.ShapeDtypeStruct((nblk * EBLK, PW), jnp.float32),
        ],
        input_output_aliases={12: 0},
    )(g2, g2, edge_attr, w1ea, w1r_row, w2e, b2e_row,
      wc1, bc1_row, wc2_row, wee, bee_row, emb_in)


def _scatter_body(c0, ncap, pay_hbm, sd_hbm, zeros_hbm, agg_hbm,
                  shared, idx_all, b0, b1, ps0, ps1, ss0, ss1):
    bufs = (b0, b1)
    psem = (ps0, ps1)
    ssem = (ss0, ss1)
    cid = lax.axis_index("c")
    sid = lax.axis_index("s")
    wid = sid * NC + cid

    rbase = pl.multiple_of(
        jnp.minimum(sid * ROWS_PER_TILE, N_NODES - ROWS_PER_TILE), 8)
    pltpu.sync_copy(zeros_hbm.at[pl.ds(rbase, ROWS_PER_TILE)],
                    shared.at[pl.ds(rbase, ROWS_PER_TILE)])

    cbase = wid * CPT_S
    valid = jnp.clip(ncap - cbase, 0, CPT_S)
    pltpu.sync_copy(sd_hbm.at[pl.ds(c0 + cbase, CPT_S)], idx_all)
    plsc.subcore_barrier()

    def group(g, _):
        for i in range(NBUF):
            j = g * NBUF + i

            @pl.when(j < valid)
            def _():
                @pl.when(g > 0)
                def _():
                    pltpu.make_async_copy(
                        bufs[i], shared.at[idx_all.at[j - NBUF]],
                        ssem[i]).wait()

                off = (cbase + j) * CH
                pltpu.async_copy(pay_hbm.at[pl.ds(off, CH)], bufs[i], psem[i])

        for i in range(NBUF):
            j = g * NBUF + i

            @pl.when(j < valid)
            def _():
                off = (cbase + j) * CH
                pltpu.make_async_copy(
                    pay_hbm.at[pl.ds(off, CH)], bufs[i], psem[i]).wait()
                pltpu.async_copy(bufs[i], shared.at[idx_all.at[j]],
                                 ssem[i], add=True)

        return _

    lax.fori_loop(0, CPT_S // NBUF, group, None)

    for i in range(NBUF):
        n_i = (valid + (NBUF - 1) - i) // NBUF

        @pl.when(valid > i)
        def _():
            last_j = (n_i - 1) * NBUF + i
            pltpu.make_async_copy(
                bufs[i], shared.at[idx_all.at[last_j]], ssem[i]).wait()

    plsc.subcore_barrier()
    pltpu.sync_copy(shared.at[pl.ds(rbase, ROWS_PER_TILE)],
                    agg_hbm.at[pl.ds(cid * N_NODES + rbase, ROWS_PER_TILE)])


def _scatter(payload, sd, zeros_tbl, c0, ncap):
    mesh = plsc.VectorSubcoreMesh(core_axis_name="c", subcore_axis_name="s")
    k = functools.partial(
        pl.kernel,
        mesh=mesh,
        out_type=jax.ShapeDtypeStruct((NC * N_NODES, PW), jnp.float32),
        scratch_types=[
            pltpu.MemorySpace.VMEM_SHARED((N_NODES, PW), jnp.float32),
            pltpu.VMEM((CPT_S, CH), jnp.int32),
        ] + [pltpu.VMEM((CH, PW), jnp.float32)] * NBUF
          + [pltpu.SemaphoreType.DMA] * (2 * NBUF),
    )(functools.partial(_scatter_body, c0, ncap))
    return k(payload, sd, zeros_tbl)


def _node_body(nf_ref, coords_ref, agga_ref, aggb_ref, batch_ref,
               w1na_ref, w1nb_ref, w1nc_ref, b1n_ref, w2n_ref, b2n_ref,
               wcm_ref, bcm_ref, wne_ref, bne_ref,
               nemb_ref, gemb_ref, cout_ref):
    agg = (agga_ref[0:N_NODES, :] + agga_ref[N_NODES:2 * N_NODES, :]
           + aggb_ref[0:N_NODES, :] + aggb_ref[N_NODES:2 * N_NODES, :])
    agg_e = agg[:, :D_EDGE]
    agg_c = agg[:, D_EDGE:D_EDGE + 3]
    c2 = coords_ref[...] + agg_c
    coord_out = (c2[:, 0:1] * wcm_ref[0:1, :]
                 + c2[:, 1:2] * wcm_ref[1:2, :]
                 + c2[:, 2:3] * wcm_ref[2:3, :]
                 + bcm_ref[...])
    pre = (jnp.dot(nf_ref[...], w1na_ref[...],
                   preferred_element_type=jnp.float32)
           + jnp.dot(agg_e, w1nb_ref[...],
                     preferred_element_type=jnp.float32)
           + coord_out[:, 0:1] * w1nc_ref[0:1, :]
           + coord_out[:, 1:2] * w1nc_ref[1:2, :]
           + coord_out[:, 2:3] * w1nc_ref[2:3, :]
           + b1n_ref[...])
    h = jnp.maximum(pre, 0.0)
    node_out = jnp.dot(h, w2n_ref[...],
                       preferred_element_type=jnp.float32) + b2n_ref[...]
    node_emb = jnp.dot(node_out, wne_ref[...],
                       preferred_element_type=jnp.float32) + bne_ref[...]
    nemb_ref[...] = node_emb
    cout_ref[...] = coord_out
    gids = lax.broadcasted_iota(jnp.int32, (N_GRAPHS, N_NODES), 0)
    onehot_t = (gids == batch_ref[...]).astype(jnp.float32)
    sums = jnp.dot(onehot_t, node_emb, preferred_element_type=jnp.float32)
    counts = jnp.sum(onehot_t, axis=1, keepdims=True)
    gemb_ref[...] = sums / jnp.maximum(counts, 1.0)


def _node(nf, coords, agg_a, agg_b, batch_row, w1na, w1nb, w1nc, b1n_row,
          w2n, b2n_row, wcm, bcm_row, wne, bne_row):
    return pl.pallas_call(
        _node_body,
        out_shape=[
            jax.ShapeDtypeStruct((N_NODES, EMB), jnp.float32),
            jax.ShapeDtypeStruct((N_GRAPHS, EMB), jnp.float32),
            jax.ShapeDtypeStruct((N_NODES, 3), jnp.float32),
        ],
    )(nf, coords, agg_a, agg_b, batch_row, w1na, w1nb, w1nc, b1n_row,
      w2n, b2n_row, wcm, bcm_row, wne, bne_row)


def kernel(node_feats, edge_index, edge_attr, coords, node_batch_vec,
           W1n, b1n, W2n, b2n, W1e, b1e, W2e, b2e,
           Wc1, bc1, Wc2, Wcm, bcm, Wne, bne, Wee, bee):
    w1es = W1e[:D_NODE]
    w1ed = W1e[D_NODE:2 * D_NODE]
    w1ea = W1e[2 * D_NODE:2 * D_NODE + D_EDGE]
    w1r_row = W1e[2 * D_NODE + D_EDGE:]

    sd = jnp.pad(edge_index.reshape(2, N_CHUNKS, CH),
                 ((0, 0), (0, SD_SIDE - N_CHUNKS), (0, 0))
                 ).reshape(2 * SD_SIDE, CH)

    t2 = _prep(node_feats, coords, w1es, w1ed, b1e.reshape(1, H))
    g2a = _gather(t2, sd, C0_A, NCAP_A)
    g2b = _gather(t2, sd, C0_B, NCAP_B)

    eargs = (w1ea, w1r_row, W2e, b2e.reshape(1, D_EDGE),
             Wc1, bc1.reshape(1, 2), Wc2.reshape(1, 2),
             Wee, bee.reshape(1, EMB))
    nblk_a = NCAP_A * CH // EBLK
    nblk_b = NCAP_B * CH // EBLK
    emb0 = jnp.zeros((N_EDGES, EMB), jnp.float32)
    emb_a, pay_a = _edge(g2a, edge_attr, *eargs, emb0, nblk_a, 0)
    emb, pay_b = _edge(g2b, edge_attr, *eargs, emb_a, nblk_b, nblk_a)

    zeros_tbl = jnp.zeros((N_NODES, PW), jnp.float32)
    agg_a = _scatter(pay_a, sd, zeros_tbl, C0_A, NCAP_A)
    agg_b = _scatter(pay_b, sd, zeros_tbl, C0_B, NCAP_B)
    node_emb, graph_emb, coord_out = _node(
        node_feats, coords, agg_a, agg_b, node_batch_vec.reshape(1, N_NODES),
        W1n[:D_NODE], W1n[D_NODE:D_NODE + D_EDGE],
        W1n[D_NODE + D_EDGE:], b1n.reshape(1, H),
        W2n, b2n.reshape(1, OUT), Wcm, bcm.reshape(1, 3),
        Wne, bne.reshape(1, EMB))
    return (node_emb, emb, graph_emb, coord_out)


# speedup vs baseline: 7.0554x; 1.0520x over previous
import functools

import jax
import jax.numpy as jnp
from jax import lax
from jax.experimental import pallas as pl
from jax.experimental.pallas import tpu as pltpu
from jax.experimental.pallas import tpu_sc as plsc

N_NODES = 10000
N_EDGES = 320000
D_NODE = 128
D_EDGE = 16
H = 64
OUT = 64
EMB = 32
N_GRAPHS = 64

TW = 128
PW = 128
CH = 128
NC = 2
NS = 16
NW = NC * NS
N_CHUNKS = N_EDGES // CH
SD_SIDE = 2560
C0_A, NCAP_A = 0, 1280
C0_B, NCAP_B = 1280, 1220
CPT_G = 80
CPT_S = 40
NBUF = 2
ROWS_PER_TILE = 632

EBLK = 2560


def _prep_body(nf_ref, coords_ref, w1es_ref, w1ed_ref, b1e_ref, t2_ref):
    nf = nf_ref[...]
    coords = coords_ref[...]
    pad = jnp.zeros((N_NODES, TW - H - 3), jnp.float32)
    p = jnp.dot(nf, w1es_ref[...], preferred_element_type=jnp.float32)
    p = p + b1e_ref[...]
    q = jnp.dot(nf, w1ed_ref[...], preferred_element_type=jnp.float32)
    t2_ref[0:N_NODES, :] = jnp.concatenate([p, coords, pad], axis=1)
    t2_ref[N_NODES:2 * N_NODES, :] = jnp.concatenate([q, coords, pad], axis=1)


def _prep(nf, coords, w1es, w1ed, b1e_row):
    return pl.pallas_call(
        _prep_body,
        out_shape=jax.ShapeDtypeStruct((2 * N_NODES, TW), jnp.float32),
    )(nf, coords, w1es, w1ed, b1e_row)


def _gather_body(c0, ncap, t2_hbm, sd_hbm, g2_hbm,
                 shared_t, idx_all, b0, b1, gs0, gs1, ws0, ws1):
    bufs = (b0, b1)
    gsem = (gs0, gs1)
    wsem = (ws0, ws1)
    cid = lax.axis_index("c")
    sid = lax.axis_index("s")

    rbase = pl.multiple_of(
        jnp.minimum(sid * ROWS_PER_TILE, N_NODES - ROWS_PER_TILE), 8)
    pltpu.sync_copy(t2_hbm.at[pl.ds(cid * N_NODES + rbase, ROWS_PER_TILE)],
                    shared_t.at[pl.ds(rbase, ROWS_PER_TILE)])

    cbase = sid * CPT_G
    valid = jnp.clip(ncap - cbase, 0, CPT_G)
    irow0 = cid * SD_SIDE + c0 + cbase
    pltpu.sync_copy(sd_hbm.at[pl.ds(irow0, CPT_G)], idx_all)
    plsc.subcore_barrier()

    ebase = cid * (ncap * CH)

    def group(g, _):
        for i in range(NBUF):
            j = g * NBUF + i

            @pl.when(j < valid)
            def _():
                @pl.when(g > 0)
                def _():
                    prev = ebase + (cbase + j - NBUF) * CH
                    pltpu.make_async_copy(
                        bufs[i], g2_hbm.at[pl.ds(prev, CH)], wsem[i]).wait()

                pltpu.async_copy(shared_t.at[idx_all.at[j]], bufs[i], gsem[i])

        for i in range(NBUF):
            j = g * NBUF + i

            @pl.when(j < valid)
            def _():
                pltpu.make_async_copy(
                    shared_t.at[idx_all.at[j]], bufs[i], gsem[i]).wait()
                off = ebase + (cbase + j) * CH
                pltpu.async_copy(bufs[i], g2_hbm.at[pl.ds(off, CH)], wsem[i])

        return _

    lax.fori_loop(0, CPT_G // NBUF, group, None)

    for i in range(NBUF):
        n_i = (valid + (NBUF - 1) - i) // NBUF

        @pl.when(valid > i)
        def _():
            last = ebase + (cbase + (n_i - 1) * NBUF + i) * CH
            pltpu.make_async_copy(
                bufs[i], g2_hbm.at[pl.ds(last, CH)], wsem[i]).wait()


def _gather(t2, sd, c0, ncap):
    mesh = plsc.VectorSubcoreMesh(core_axis_name="c", subcore_axis_name="s")
    k = functools.partial(
        pl.kernel,
        mesh=mesh,
        out_type=jax.ShapeDtypeStruct((2 * ncap * CH, TW), jnp.float32),
        scratch_types=[
            pltpu.MemorySpace.VMEM_SHARED((N_NODES, TW), jnp.float32),
            pltpu.VMEM((CPT_G, CH), jnp.int32),
        ] + [pltpu.VMEM((CH, TW), jnp.float32)] * NBUF
          + [pltpu.SemaphoreType.DMA] * (2 * NBUF),
    )(functools.partial(_gather_body, c0, ncap))
    return k(t2, sd)


def _edge_body(gs_ref, gd_ref, ea_ref, w1ea_ref, w1r_ref, w2e_ref, b2e_ref,
               wc1_ref, bc1_ref, wc2_ref, wee_ref, bee_ref,
               emb_ref, pay_ref):
    gs = gs_ref[...]
    gd = gd_ref[...]
    bond = gs[:, H:H + 3] - gd[:, H:H + 3]
    radial = jnp.sum(bond * bond, axis=1, keepdims=True)
    pre = (gs[:, :H] + gd[:, :H]
           + jnp.dot(ea_ref[...], w1ea_ref[...],
                     preferred_element_type=jnp.float32)
           + radial * w1r_ref[...])
    hidden = jnp.maximum(pre, 0.0)
    edge_out = jnp.dot(hidden, w2e_ref[...],
                       preferred_element_type=jnp.float32) + b2e_ref[...]
    t = jnp.maximum(
        jnp.dot(edge_out, wc1_ref[...], preferred_element_type=jnp.float32)
        + bc1_ref[...], 0.0)
    e_c = t[:, 0:1] * wc2_ref[0, 0] + t[:, 1:2] * wc2_ref[0, 1]
    bond_n = bond / (jnp.sqrt(radial) + 1.0)
    trans = bond_n * e_c
    emb_ref[...] = jnp.dot(edge_out, wee_ref[...],
                           preferred_element_type=jnp.float32) + bee_ref[...]
    pad = jnp.zeros((EBLK, PW - D_EDGE - 3), jnp.float32)
    pay_ref[...] = jnp.concatenate([edge_out, trans, pad], axis=1)


def _edge(g2, edge_attr, w1ea, w1r_row, w2e, b2e_row,
          wc1, bc1_row, wc2_row, wee, bee_row, nblk, blk0):
    full = lambda s: pl.BlockSpec(s, lambda i: (0, 0))
    return pl.pallas_call(
        _edge_body,
        grid=(nblk,),
        in_specs=[pl.BlockSpec((EBLK, TW), lambda i: (i, 0)),
                  pl.BlockSpec((EBLK, TW), lambda i, n=nblk: (n + i, 0)),
                  pl.BlockSpec((EBLK, D_EDGE), lambda i, b=blk0: (b + i, 0)),
                  full((D_EDGE, H)), full((1, H)), full((H, D_EDGE)),
                  full((1, D_EDGE)), full((D_EDGE, 2)), full((1, 2)),
                  full((1, 2)), full((D_EDGE, EMB)), full((1, EMB))],
        out_specs=[pl.BlockSpec((EBLK, EMB), lambda i: (i, 0)),
                   pl.BlockSpec((EBLK, PW), lambda i: (i, 0))],
        out_shape=[
            jax.ShapeDtypeStruct((nblk * EBLK, EMB), jnp.float32),
            jax.ShapeDtypeStruct((nblk * EBLK, PW), jnp.float32),
        ],
    )(g2, g2, edge_attr, w1ea, w1r_row, w2e, b2e_row,
      wc1, bc1_row, wc2_row, wee, bee_row)


def _scatter_body(c0, ncap, pay_hbm, sd_hbm, zeros_hbm, agg_hbm,
                  shared, idx_all, b0, b1, ps0, ps1, ss0, ss1):
    bufs = (b0, b1)
    psem = (ps0, ps1)
    ssem = (ss0, ss1)
    cid = lax.axis_index("c")
    sid = lax.axis_index("s")
    wid = sid * NC + cid

    rbase = pl.multiple_of(
        jnp.minimum(sid * ROWS_PER_TILE, N_NODES - ROWS_PER_TILE), 8)
    pltpu.sync_copy(zeros_hbm.at[pl.ds(rbase, ROWS_PER_TILE)],
                    shared.at[pl.ds(rbase, ROWS_PER_TILE)])

    cbase = wid * CPT_S
    valid = jnp.clip(ncap - cbase, 0, CPT_S)
    pltpu.sync_copy(sd_hbm.at[pl.ds(c0 + cbase, CPT_S)], idx_all)
    plsc.subcore_barrier()

    def group(g, _):
        for i in range(NBUF):
            j = g * NBUF + i

            @pl.when(j < valid)
            def _():
                @pl.when(g > 0)
                def _():
                    pltpu.make_async_copy(
                        bufs[i], shared.at[idx_all.at[j - NBUF]],
                        ssem[i]).wait()

                off = (cbase + j) * CH
                pltpu.async_copy(pay_hbm.at[pl.ds(off, CH)], bufs[i], psem[i])

        for i in range(NBUF):
            j = g * NBUF + i

            @pl.when(j < valid)
            def _():
                off = (cbase + j) * CH
                pltpu.make_async_copy(
                    pay_hbm.at[pl.ds(off, CH)], bufs[i], psem[i]).wait()
                pltpu.async_copy(bufs[i], shared.at[idx_all.at[j]],
                                 ssem[i], add=True)

        return _

    lax.fori_loop(0, CPT_S // NBUF, group, None)

    for i in range(NBUF):
        n_i = (valid + (NBUF - 1) - i) // NBUF

        @pl.when(valid > i)
        def _():
            last_j = (n_i - 1) * NBUF + i
            pltpu.make_async_copy(
                bufs[i], shared.at[idx_all.at[last_j]], ssem[i]).wait()

    plsc.subcore_barrier()
    pltpu.sync_copy(shared.at[pl.ds(rbase, ROWS_PER_TILE)],
                    agg_hbm.at[pl.ds(cid * N_NODES + rbase, ROWS_PER_TILE)])


def _scatter(payload, sd, zeros_tbl, c0, ncap):
    mesh = plsc.VectorSubcoreMesh(core_axis_name="c", subcore_axis_name="s")
    k = functools.partial(
        pl.kernel,
        mesh=mesh,
        out_type=jax.ShapeDtypeStruct((NC * N_NODES, PW), jnp.float32),
        scratch_types=[
            pltpu.MemorySpace.VMEM_SHARED((N_NODES, PW), jnp.float32),
            pltpu.VMEM((CPT_S, CH), jnp.int32),
        ] + [pltpu.VMEM((CH, PW), jnp.float32)] * NBUF
          + [pltpu.SemaphoreType.DMA] * (2 * NBUF),
    )(functools.partial(_scatter_body, c0, ncap))
    return k(payload, sd, zeros_tbl)


def _node_body(nf_ref, coords_ref, agga_ref, aggb_ref, batch_ref,
               w1na_ref, w1nb_ref, w1nc_ref, b1n_ref, w2n_ref, b2n_ref,
               wcm_ref, bcm_ref, wne_ref, bne_ref,
               nemb_ref, gemb_ref, cout_ref):
    agg = (agga_ref[0:N_NODES, :] + agga_ref[N_NODES:2 * N_NODES, :]
           + aggb_ref[0:N_NODES, :] + aggb_ref[N_NODES:2 * N_NODES, :])
    agg_e = agg[:, :D_EDGE]
    agg_c = agg[:, D_EDGE:D_EDGE + 3]
    c2 = coords_ref[...] + agg_c
    coord_out = (c2[:, 0:1] * wcm_ref[0:1, :]
                 + c2[:, 1:2] * wcm_ref[1:2, :]
                 + c2[:, 2:3] * wcm_ref[2:3, :]
                 + bcm_ref[...])
    pre = (jnp.dot(nf_ref[...], w1na_ref[...],
                   preferred_element_type=jnp.float32)
           + jnp.dot(agg_e, w1nb_ref[...],
                     preferred_element_type=jnp.float32)
           + coord_out[:, 0:1] * w1nc_ref[0:1, :]
           + coord_out[:, 1:2] * w1nc_ref[1:2, :]
           + coord_out[:, 2:3] * w1nc_ref[2:3, :]
           + b1n_ref[...])
    h = jnp.maximum(pre, 0.0)
    node_out = jnp.dot(h, w2n_ref[...],
                       preferred_element_type=jnp.float32) + b2n_ref[...]
    node_emb = jnp.dot(node_out, wne_ref[...],
                       preferred_element_type=jnp.float32) + bne_ref[...]
    nemb_ref[...] = node_emb
    cout_ref[...] = coord_out
    gids = lax.broadcasted_iota(jnp.int32, (N_GRAPHS, N_NODES), 0)
    onehot_t = (gids == batch_ref[...]).astype(jnp.float32)
    sums = jnp.dot(onehot_t, node_emb, preferred_element_type=jnp.float32)
    counts = jnp.sum(onehot_t, axis=1, keepdims=True)
    gemb_ref[...] = sums / jnp.maximum(counts, 1.0)


def _node(nf, coords, agg_a, agg_b, batch_row, w1na, w1nb, w1nc, b1n_row,
          w2n, b2n_row, wcm, bcm_row, wne, bne_row):
    return pl.pallas_call(
        _node_body,
        out_shape=[
            jax.ShapeDtypeStruct((N_NODES, EMB), jnp.float32),
            jax.ShapeDtypeStruct((N_GRAPHS, EMB), jnp.float32),
            jax.ShapeDtypeStruct((N_NODES, 3), jnp.float32),
        ],
    )(nf, coords, agg_a, agg_b, batch_row, w1na, w1nb, w1nc, b1n_row,
      w2n, b2n_row, wcm, bcm_row, wne, bne_row)


def kernel(node_feats, edge_index, edge_attr, coords, node_batch_vec,
           W1n, b1n, W2n, b2n, W1e, b1e, W2e, b2e,
           Wc1, bc1, Wc2, Wcm, bcm, Wne, bne, Wee, bee):
    w1es = W1e[:D_NODE]
    w1ed = W1e[D_NODE:2 * D_NODE]
    w1ea = W1e[2 * D_NODE:2 * D_NODE + D_EDGE]
    w1r_row = W1e[2 * D_NODE + D_EDGE:]

    sd = jnp.pad(edge_index.reshape(2, N_CHUNKS, CH),
                 ((0, 0), (0, SD_SIDE - N_CHUNKS), (0, 0))
                 ).reshape(2 * SD_SIDE, CH)

    t2 = _prep(node_feats, coords, w1es, w1ed, b1e.reshape(1, H))
    g2a = _gather(t2, sd, C0_A, NCAP_A)
    g2b = _gather(t2, sd, C0_B, NCAP_B)

    eargs = (w1ea, w1r_row, W2e, b2e.reshape(1, D_EDGE),
             Wc1, bc1.reshape(1, 2), Wc2.reshape(1, 2),
             Wee, bee.reshape(1, EMB))
    nblk_a = NCAP_A * CH // EBLK
    nblk_b = NCAP_B * CH // EBLK
    emb_a, pay_a = _edge(g2a, edge_attr, *eargs, nblk_a, 0)
    emb_b, pay_b = _edge(g2b, edge_attr, *eargs, nblk_b, nblk_a)
    emb = jnp.concatenate([emb_a, emb_b], axis=0)

    zeros_tbl = jnp.zeros((N_NODES, PW), jnp.float32)
    agg_a = _scatter(pay_a, sd, zeros_tbl, C0_A, NCAP_A)
    agg_b = _scatter(pay_b, sd, zeros_tbl, C0_B, NCAP_B)
    node_emb, graph_emb, coord_out = _node(
        node_feats, coords, agg_a, agg_b, node_batch_vec.reshape(1, N_NODES),
        W1n[:D_NODE], W1n[D_NODE:D_NODE + D_EDGE],
        W1n[D_NODE + D_EDGE:], b1n.reshape(1, H),
        W2n, b2n.reshape(1, OUT), Wcm, bcm.reshape(1, 3),
        Wne, bne.reshape(1, EMB))
    return (node_emb, emb, graph_emb, coord_out)
